# Initial kernel scaffold; baseline (speedup 1.0000x reference)
import functools

import jax
import jax.numpy as jnp
from jax import lax
from jax.experimental import pallas as pl
from jax.experimental.pallas import tpu as pltpu
from jax.experimental.pallas import tpu_sc as plsc

_B = 16384
_NC = 2
_NS = 16
_NW = _NC * _NS
_BPW = _B // _NW
_L = 16

_VALUE_RANGE = 8.0
_A_RANGE = 3.0
_D = 1.702


def _sigmoid(x):
    return 1.0 / (1.0 + jnp.exp(-x))


def _body(user_hbm, item_hbm, pair_hbm, theta_hbm, a_hbm, b_hbm, c_hbm,
          irf_hbm, th_hbm, tp_hbm,
          uidx, iidx, pidx, th_v, tp_v, a_v, b_v, c_v, irf_v, sem):
    wid = lax.axis_index("s") * _NC + lax.axis_index("c")
    base = wid * _BPW

    pltpu.sync_copy(user_hbm.at[pl.ds(base, _BPW)], uidx)
    pltpu.sync_copy(item_hbm.at[pl.ds(base, _BPW)], iidx)
    pltpu.sync_copy(pair_hbm.at[pl.ds(base, _BPW)], pidx)

    cps = (pltpu.async_copy(theta_hbm.at[uidx], th_v, sem),
           pltpu.async_copy(theta_hbm.at[pidx], tp_v, sem),
           pltpu.async_copy(a_hbm.at[iidx], a_v, sem),
           pltpu.async_copy(b_hbm.at[iidx], b_v, sem),
           pltpu.async_copy(c_hbm.at[iidx], c_v, sem))
    for cp in cps:
        cp.wait()

    def chunk(i, carry):
        sl = pl.ds(i * _L, _L)
        th_s = _sigmoid(th_v[sl])
        tp_s = _sigmoid(tp_v[sl])
        a = _A_RANGE * _sigmoid(a_v[sl])
        b = _VALUE_RANGE * (_sigmoid(b_v[sl]) - 0.5)
        c = _sigmoid(c_v[sl])
        theta = _VALUE_RANGE * (th_s - 0.5)
        irf = c + (1.0 - c) / (1.0 + jnp.exp(-_D * a * (theta - b)))
        irf_v[sl] = irf
        th_v[sl] = th_s
        tp_v[sl] = tp_s
        return carry

    lax.fori_loop(0, _BPW // _L, chunk, 0)

    pltpu.sync_copy(irf_v, irf_hbm.at[pl.ds(base, _BPW)])
    pltpu.sync_copy(th_v, th_hbm.at[pl.ds(base, _BPW)])
    pltpu.sync_copy(tp_v, tp_hbm.at[pl.ds(base, _BPW)])


_irt_sc = functools.partial(
    pl.kernel,
    mesh=plsc.VectorSubcoreMesh(core_axis_name="c", subcore_axis_name="s"),
    out_type=(jax.ShapeDtypeStruct((_B,), jnp.float32),
              jax.ShapeDtypeStruct((_B,), jnp.float32),
              jax.ShapeDtypeStruct((_B,), jnp.float32)),
    scratch_types=[
        pltpu.VMEM((_BPW,), jnp.int32),
        pltpu.VMEM((_BPW,), jnp.int32),
        pltpu.VMEM((_BPW,), jnp.int32),
        pltpu.VMEM((_BPW,), jnp.float32),
        pltpu.VMEM((_BPW,), jnp.float32),
        pltpu.VMEM((_BPW,), jnp.float32),
        pltpu.VMEM((_BPW,), jnp.float32),
        pltpu.VMEM((_BPW,), jnp.float32),
        pltpu.VMEM((_BPW,), jnp.float32),
        pltpu.SemaphoreType.DMA,
    ],
)(_body)


def kernel(user, item, user_pair, theta_w, a_w, b_w, c_w):
    theta_f = theta_w.reshape(-1)
    a_f = a_w.reshape(-1)
    b_f = b_w.reshape(-1)
    c_f = c_w.reshape(-1)
    return _irt_sc(user, item, user_pair, theta_f, a_f, b_f, c_f)


# speedup vs baseline: 1.3579x; 1.3579x over previous
import functools

import jax
import jax.numpy as jnp
from jax import lax
from jax.experimental import pallas as pl
from jax.experimental.pallas import tpu as pltpu
from jax.experimental.pallas import tpu_sc as plsc

_B = 16384
_NC = 2
_NS = 16
_NW = _NC * _NS
_BPW = _B // _NW
_L = 16
_IDX_CHUNK = 128

_VALUE_RANGE = 8.0
_A_RANGE = 3.0
_D = 1.702


def _sigmoid(x):
    return 1.0 / (1.0 + jnp.exp(-x))


def _body(user_hbm, item_hbm, pair_hbm, theta_hbm, a_hbm, b_hbm, c_hbm,
          irf_hbm, th_hbm, tp_hbm,
          uidx, iidx, pidx, th_v, tp_v, a_v, b_v, c_v, irf_v, sem):
    wid = lax.axis_index("s") * _NC + lax.axis_index("c")
    base = wid * _BPW

    pltpu.sync_copy(user_hbm.at[pl.ds(base, _BPW)], uidx)
    pltpu.sync_copy(item_hbm.at[pl.ds(base, _BPW)], iidx)
    pltpu.sync_copy(pair_hbm.at[pl.ds(base, _BPW)], pidx)

    cps = []
    for j in range(_BPW // _IDX_CHUNK):
        sl = pl.ds(j * _IDX_CHUNK, _IDX_CHUNK)
        cps.append(pltpu.async_copy(theta_hbm.at[uidx.at[sl]], th_v.at[sl], sem))
        cps.append(pltpu.async_copy(theta_hbm.at[pidx.at[sl]], tp_v.at[sl], sem))
        cps.append(pltpu.async_copy(a_hbm.at[iidx.at[sl]], a_v.at[sl], sem))
        cps.append(pltpu.async_copy(b_hbm.at[iidx.at[sl]], b_v.at[sl], sem))
        cps.append(pltpu.async_copy(c_hbm.at[iidx.at[sl]], c_v.at[sl], sem))
    for cp in cps:
        cp.wait()

    def chunk(i, carry):
        sl = pl.ds(i * _L, _L)
        theta = _VALUE_RANGE * (_sigmoid(th_v[sl]) - 0.5)
        theta_pair = _VALUE_RANGE * (_sigmoid(tp_v[sl]) - 0.5)
        a = _A_RANGE * _sigmoid(a_v[sl])
        b = _VALUE_RANGE * (_sigmoid(b_v[sl]) - 0.5)
        c = _sigmoid(c_v[sl])
        irf = c + (1.0 - c) / (1.0 + jnp.exp(-_D * a * (theta - b)))
        irf_v[sl] = irf
        th_v[sl] = _sigmoid(theta)
        tp_v[sl] = _sigmoid(theta_pair)
        return carry

    lax.fori_loop(0, _BPW // _L, chunk, 0)

    pltpu.sync_copy(irf_v, irf_hbm.at[pl.ds(base, _BPW)])
    pltpu.sync_copy(th_v, th_hbm.at[pl.ds(base, _BPW)])
    pltpu.sync_copy(tp_v, tp_hbm.at[pl.ds(base, _BPW)])


_irt_sc = functools.partial(
    pl.kernel,
    mesh=plsc.VectorSubcoreMesh(core_axis_name="c", subcore_axis_name="s"),
    out_type=(jax.ShapeDtypeStruct((_B,), jnp.float32),
              jax.ShapeDtypeStruct((_B,), jnp.float32),
              jax.ShapeDtypeStruct((_B,), jnp.float32)),
    scratch_types=[
        pltpu.VMEM((_BPW,), jnp.int32),
        pltpu.VMEM((_BPW,), jnp.int32),
        pltpu.VMEM((_BPW,), jnp.int32),
        pltpu.VMEM((_BPW,), jnp.float32),
        pltpu.VMEM((_BPW,), jnp.float32),
        pltpu.VMEM((_BPW,), jnp.float32),
        pltpu.VMEM((_BPW,), jnp.float32),
        pltpu.VMEM((_BPW,), jnp.float32),
        pltpu.VMEM((_BPW,), jnp.float32),
        pltpu.SemaphoreType.DMA,
    ],
)(_body)


def kernel(user, item, user_pair, theta_w, a_w, b_w, c_w):
    theta_f = theta_w.reshape(-1)
    a_f = a_w.reshape(-1)
    b_f = b_w.reshape(-1)
    c_f = c_w.reshape(-1)
    return _irt_sc(user, item, user_pair, theta_f, a_f, b_f, c_f)


# speedup vs baseline: 1.3747x; 1.0123x over previous
import functools

import jax
import jax.numpy as jnp
from jax import lax
from jax.experimental import pallas as pl
from jax.experimental.pallas import tpu as pltpu
from jax.experimental.pallas import tpu_sc as plsc

_B = 16384
_NC = 2
_NS = 16
_NW = _NC * _NS
_BPW = _B // _NW
_L = 16
_IDX_CHUNK = 128

_VALUE_RANGE = 8.0
_A_RANGE = 3.0
_D = 1.702


def _sigmoid(x):
    return 1.0 / (1.0 + jnp.exp(-x))


def _body(user_hbm, item_hbm, pair_hbm, theta_hbm, a_hbm, b_hbm, c_hbm,
          irf_hbm, th_hbm, tp_hbm,
          uidx, iidx, pidx, th_v, tp_v, a_v, b_v, c_v, irf_v, sem):
    wid = lax.axis_index("s") * _NC + lax.axis_index("c")
    base = wid * _BPW

    icps = (pltpu.async_copy(user_hbm.at[pl.ds(base, _BPW)], uidx, sem),
            pltpu.async_copy(item_hbm.at[pl.ds(base, _BPW)], iidx, sem),
            pltpu.async_copy(pair_hbm.at[pl.ds(base, _BPW)], pidx, sem))
    for cp in icps:
        cp.wait()
    cps = (pltpu.async_copy(theta_hbm.at[uidx], th_v, sem),
           pltpu.async_copy(theta_hbm.at[pidx], tp_v, sem),
           pltpu.async_copy(a_hbm.at[iidx], a_v, sem),
           pltpu.async_copy(b_hbm.at[iidx], b_v, sem),
           pltpu.async_copy(c_hbm.at[iidx], c_v, sem))
    for cp in cps:
        cp.wait()

    def chunk(i, carry):
        sl = pl.ds(i * _L, _L)
        theta = _VALUE_RANGE * (_sigmoid(th_v[sl]) - 0.5)
        theta_pair = _VALUE_RANGE * (_sigmoid(tp_v[sl]) - 0.5)
        a = _A_RANGE * _sigmoid(a_v[sl])
        b = _VALUE_RANGE * (_sigmoid(b_v[sl]) - 0.5)
        c = _sigmoid(c_v[sl])
        irf = c + (1.0 - c) / (1.0 + jnp.exp(-_D * a * (theta - b)))
        irf_v[sl] = irf
        th_v[sl] = _sigmoid(theta)
        tp_v[sl] = _sigmoid(theta_pair)
        return carry

    lax.fori_loop(0, _BPW // _L, chunk, 0)

    ocps = (pltpu.async_copy(irf_v, irf_hbm.at[pl.ds(base, _BPW)], sem),
            pltpu.async_copy(th_v, th_hbm.at[pl.ds(base, _BPW)], sem),
            pltpu.async_copy(tp_v, tp_hbm.at[pl.ds(base, _BPW)], sem))
    for cp in ocps:
        cp.wait()


_irt_sc = functools.partial(
    pl.kernel,
    mesh=plsc.VectorSubcoreMesh(core_axis_name="c", subcore_axis_name="s"),
    out_type=(jax.ShapeDtypeStruct((_B,), jnp.float32),
              jax.ShapeDtypeStruct((_B,), jnp.float32),
              jax.ShapeDtypeStruct((_B,), jnp.float32)),
    scratch_types=[
        pltpu.VMEM((_BPW,), jnp.int32),
        pltpu.VMEM((_BPW,), jnp.int32),
        pltpu.VMEM((_BPW,), jnp.int32),
        pltpu.VMEM((_BPW,), jnp.float32),
        pltpu.VMEM((_BPW,), jnp.float32),
        pltpu.VMEM((_BPW,), jnp.float32),
        pltpu.VMEM((_BPW,), jnp.float32),
        pltpu.VMEM((_BPW,), jnp.float32),
        pltpu.VMEM((_BPW,), jnp.float32),
        pltpu.SemaphoreType.DMA,
    ],
)(_body)


def kernel(user, item, user_pair, theta_w, a_w, b_w, c_w):
    theta_f = theta_w.reshape(-1)
    a_f = a_w.reshape(-1)
    b_f = b_w.reshape(-1)
    c_f = c_w.reshape(-1)
    return _irt_sc(user, item, user_pair, theta_f, a_f, b_f, c_f)


# speedup vs baseline: 2.7593x; 2.0072x over previous
import functools

import jax
import jax.numpy as jnp
from jax import lax
from jax.experimental import pallas as pl
from jax.experimental.pallas import tpu as pltpu
from jax.experimental.pallas import tpu_sc as plsc

_B = 16384
_NC = 2
_NS = 16
_NW = _NC * _NS
_BPW = _B // _NW
_L = 16

_VALUE_RANGE = 8.0
_A_RANGE = 3.0
_D = 1.702


def _sigmoid(x):
    return 1.0 / (1.0 + jnp.exp(-x))


def _flat_pad(t):
    n = t.shape[0]
    pad = (-n) % 1024
    return jnp.pad(t, ((0, pad), (0, 0))).reshape(-1)


def _body(user_hbm, item_hbm, pair_hbm, theta_hbm, a_hbm, b_hbm, c_hbm,
          irf_hbm, th_hbm, tp_hbm,
          uidx, iidx, pidx, th_v, tp_v, a_v, b_v, c_v,
          irf_o, th_o, tp_o, sem):
    wid = lax.axis_index("s") * _NC + lax.axis_index("c")
    base = wid * _BPW

    icps = (pltpu.async_copy(user_hbm.at[pl.ds(base, _BPW)], uidx, sem),
            pltpu.async_copy(item_hbm.at[pl.ds(base, _BPW)], iidx, sem),
            pltpu.async_copy(pair_hbm.at[pl.ds(base, _BPW)], pidx, sem))
    for cp in icps:
        cp.wait()
    cps = (pltpu.async_copy(theta_hbm.at[uidx], th_v, sem),
           pltpu.async_copy(theta_hbm.at[pidx], tp_v, sem),
           pltpu.async_copy(a_hbm.at[iidx], a_v, sem),
           pltpu.async_copy(b_hbm.at[iidx], b_v, sem),
           pltpu.async_copy(c_hbm.at[iidx], c_v, sem))
    for cp in cps:
        cp.wait()

    def chunk(i, carry):
        sl = pl.ds(i * _L, _L)
        theta = _VALUE_RANGE * (_sigmoid(th_v[sl]) - 0.5)
        theta_pair = _VALUE_RANGE * (_sigmoid(tp_v[sl]) - 0.5)
        a = _A_RANGE * _sigmoid(a_v[sl])
        b = _VALUE_RANGE * (_sigmoid(b_v[sl]) - 0.5)
        c = _sigmoid(c_v[sl])
        irf = c + (1.0 - c) / (1.0 + jnp.exp(-_D * a * (theta - b)))
        irf_o[sl] = irf
        th_o[sl] = _sigmoid(theta)
        tp_o[sl] = _sigmoid(theta_pair)
        return carry

    lax.fori_loop(0, _BPW // _L, chunk, 0)

    ocps = (pltpu.async_copy(irf_o, irf_hbm.at[pl.ds(base, _BPW)], sem),
            pltpu.async_copy(th_o, th_hbm.at[pl.ds(base, _BPW)], sem),
            pltpu.async_copy(tp_o, tp_hbm.at[pl.ds(base, _BPW)], sem))
    for cp in ocps:
        cp.wait()


_irt_sc = functools.partial(
    pl.kernel,
    mesh=plsc.VectorSubcoreMesh(core_axis_name="c", subcore_axis_name="s"),
    out_type=(jax.ShapeDtypeStruct((_B,), jnp.float32),
              jax.ShapeDtypeStruct((_B,), jnp.float32),
              jax.ShapeDtypeStruct((_B,), jnp.float32)),
    scratch_types=[
        pltpu.VMEM((_BPW,), jnp.int32),
        pltpu.VMEM((_BPW,), jnp.int32),
        pltpu.VMEM((_BPW,), jnp.int32),
        pltpu.VMEM((_BPW,), jnp.float32),
        pltpu.VMEM((_BPW,), jnp.float32),
        pltpu.VMEM((_BPW,), jnp.float32),
        pltpu.VMEM((_BPW,), jnp.float32),
        pltpu.VMEM((_BPW,), jnp.float32),
        pltpu.VMEM((_BPW,), jnp.float32),
        pltpu.VMEM((_BPW,), jnp.float32),
        pltpu.VMEM((_BPW,), jnp.float32),
        pltpu.SemaphoreType.DMA,
    ],
)(_body)


def kernel(user, item, user_pair, theta_w, a_w, b_w, c_w):
    return _irt_sc(user, item, user_pair,
                   _flat_pad(theta_w), _flat_pad(a_w),
                   _flat_pad(b_w), _flat_pad(c_w))
